# baseline (speedup 1.0000x reference)
import functools
import jax
import jax.numpy as jnp
from jax import lax
from jax.experimental import pallas as pl
from jax.experimental.pallas import tpu as pltpu
from jax.experimental.pallas import tpu_sc as plsc

VOCAB = 511
DIM = 10
MAX_POS = 16
EPS = 1e-12
B = 16384
L = 16
VPAD = 512
TROWS = VPAD * L
TWORDS = TROWS * DIM

NC = 2
NS = 16
NW = NC * NS
ROWS = B * L
RPW = ROWS // NW
BPW = B // NW
IDC = 32
VPW = VPAD // NW
QROWS = 128
NQ = RPW // QROWS


def _rsqrt_newton(a):
    i = plsc.bitcast(a, jnp.int32)
    i = jnp.int32(0x5F3759DF) - lax.shift_right_logical(i, 1)
    y = plsc.bitcast(i, jnp.float32)
    half = a * 0.5
    for _ in range(3):
        y = y * (1.5 - half * y * y)
    return y


def _sc_table_body(w_hbm, p_hbm, g_hbm, b_hbm, tab_hbm, w_v, p_v, g_v, b_v,
                   stage_v, sem):
    wid = lax.axis_index("s") * NC + lax.axis_index("c")
    v0 = wid * VPW
    pltpu.async_copy(w_hbm.at[pl.ds(v0, VPW), :], w_v, sem).wait()
    pltpu.async_copy(p_hbm, p_v, sem).wait()
    pltpu.async_copy(g_hbm, g_v, sem).wait()
    pltpu.async_copy(b_hbm, b_v, sem).wait()
    liota = lax.iota(jnp.int32, 16)

    pcol = [plsc.load_gather(p_v, [liota, jnp.full((16,), c, jnp.int32)])
            for c in range(DIM)]
    gcol = [plsc.load_gather(g_v, [liota, jnp.full((16,), c, jnp.int32)])
            for c in range(DIM)]
    bcol = [plsc.load_gather(b_v, [liota, jnp.full((16,), c, jnp.int32)])
            for c in range(DIM)]

    def group_body(g, carry):
        gsplat = jnp.full((16,), g, jnp.int32)
        x = []
        for c in range(DIM):
            wsc = plsc.load_gather(w_v, [gsplat, jnp.full((16,), c, jnp.int32)])
            x.append(wsc + pcol[c])
        s = x[0]
        for c in range(1, DIM):
            s = s + x[c]
        mean = s * (1.0 / DIM)
        d0 = x[0] - mean
        s2 = d0 * d0
        for c in range(1, DIM):
            dc = x[c] - mean
            s2 = s2 + dc * dc
        inv = _rsqrt_newton(s2 * (1.0 / DIM) + EPS)
        wbase = g * (16 * DIM)
        for c in range(DIM):
            val = (x[c] - mean) * inv * gcol[c] + bcol[c]
            stage_v[pl.ds(wbase + c * 16, 16)] = val
        return carry

    lax.fori_loop(0, VPW, group_body, 0)
    pltpu.sync_copy(stage_v, tab_hbm.at[pl.ds(v0 * (16 * DIM), VPW * 16 * DIM)])


_sc_table = functools.partial(
    pl.kernel,
    out_type=jax.ShapeDtypeStruct((TWORDS,), jnp.float32),
    mesh=plsc.VectorSubcoreMesh(core_axis_name="c", subcore_axis_name="s"),
    scratch_types=[
        pltpu.VMEM((VPW, DIM), jnp.float32),
        pltpu.VMEM((MAX_POS, DIM), jnp.float32),
        pltpu.VMEM((16, DIM), jnp.float32),
        pltpu.VMEM((16, DIM), jnp.float32),
        pltpu.VMEM((VPW * 16 * DIM,), jnp.float32),
        pltpu.SemaphoreType.DMA,
    ],
    compiler_params=pltpu.CompilerParams(
        use_tc_tiling_on_sc=True, needs_layout_passes=False),
)(_sc_table_body)


def _sc_gather_body(tab_hbm, ids_hbm, out_hbm, tab_v, ids_v, idx_v, buf_v,
                    sem_t, sem_i, sem_o):
    wid = lax.axis_index("s") * NC + lax.axis_index("c")
    base = wid * RPW
    bbase = wid * BPW
    tdesc = pltpu.async_copy(tab_hbm, tab_v, sem_t)
    liota = lax.iota(jnp.int32, 16)
    liota10 = liota * DIM

    for q in range(BPW // IDC):
        pltpu.async_copy(
            ids_hbm.at[pl.ds(bbase + q * IDC, IDC), :], ids_v, sem_i).wait()

        def fidx_body(j, carry):
            idx_v[pl.ds((q * IDC + j) * 16, 16)] = \
                ids_v[j, :] * (L * DIM) + liota
            return carry

        lax.fori_loop(0, IDC, fidx_body, 0)

    dimsplat = jnp.full((16,), DIM, jnp.int32)
    pats = []
    for j in range(DIM):
        p = liota + (j * 16)
        rvec = lax.div(p, dimsplat)
        cvec = p - rvec * DIM
        pats.append((rvec, cvec, cvec * 16))

    tdesc.wait()
    descs = {}
    for h in range(NQ):
        par = h & 1
        if h >= 2:
            descs[h - 2].wait()

        def pack_body(g, carry):
            r0 = h * QROWS + g * 16
            for j in range(DIM):
                rvec, cvec, cvec16 = pats[j]
                wvec = plsc.load_gather(idx_v, [r0 + rvec]) + cvec16
                v = plsc.load_gather(tab_v, [wvec])
                plsc.store_scatter(buf_v.at[par], [g * 16 + rvec, cvec], v)
            return carry

        lax.fori_loop(0, QROWS // 16, pack_body, 0)
        descs[h] = pltpu.async_copy(
            buf_v.at[par],
            out_hbm.at[pl.ds(base + h * QROWS, QROWS), :], sem_o)
    descs[NQ - 2].wait()
    descs[NQ - 1].wait()


_sc_gather = functools.partial(
    pl.kernel,
    out_type=jax.ShapeDtypeStruct((ROWS, DIM), jnp.float32),
    mesh=plsc.VectorSubcoreMesh(core_axis_name="c", subcore_axis_name="s"),
    scratch_types=[
        pltpu.VMEM((TWORDS,), jnp.float32),
        pltpu.VMEM((IDC, L), jnp.int32),
        pltpu.VMEM((RPW,), jnp.int32),
        pltpu.VMEM((2, QROWS, DIM), jnp.float32),
        pltpu.SemaphoreType.DMA,
        pltpu.SemaphoreType.DMA,
        pltpu.SemaphoreType.DMA,
    ],
    compiler_params=pltpu.CompilerParams(
        use_tc_tiling_on_sc=True, needs_layout_passes=False),
)(_sc_gather_body)


def kernel(input_ids, word_emb, pos_emb, ln_weight, ln_bias):
    w_pad = jnp.zeros((VPAD, DIM), jnp.float32).at[:VOCAB].set(word_emb)
    g16 = jnp.tile(ln_weight.reshape(1, DIM), (16, 1))
    b16 = jnp.tile(ln_bias.reshape(1, DIM), (16, 1))
    table = _sc_table(w_pad, pos_emb, g16, b16)
    out = _sc_gather(table, input_ids)
    return out.reshape(B, L, DIM)


# speedup vs baseline: 1.0503x; 1.0503x over previous
import functools
import jax
import jax.numpy as jnp
from jax import lax
from jax.experimental import pallas as pl
from jax.experimental.pallas import tpu as pltpu
from jax.experimental.pallas import tpu_sc as plsc

VOCAB = 511
DIM = 10
MAX_POS = 16
EPS = 1e-12
B = 16384
L = 16
VPAD = 512
TROWS = VPAD * L
TWORDS = TROWS * DIM

NC = 2
NS = 16
NW = NC * NS
ROWS = B * L
RPW = ROWS // NW
BPW = B // NW
IDC = 16
VPW = VPAD // NW
QROWS = 128
NQ = RPW // QROWS
NQPC = (IDC * L) // QROWS


def _rsqrt_newton(a):
    i = plsc.bitcast(a, jnp.int32)
    i = jnp.int32(0x5F3759DF) - lax.shift_right_logical(i, 1)
    y = plsc.bitcast(i, jnp.float32)
    half = a * 0.5
    for _ in range(3):
        y = y * (1.5 - half * y * y)
    return y


def _sc_table_body(w_hbm, p_hbm, g_hbm, b_hbm, tab_hbm, w_v, p_v, g_v, b_v,
                   stage_v, sem):
    wid = lax.axis_index("s") * NC + lax.axis_index("c")
    v0 = wid * VPW
    pltpu.async_copy(w_hbm.at[pl.ds(v0, VPW), :], w_v, sem).wait()
    pltpu.async_copy(p_hbm, p_v, sem).wait()
    pltpu.async_copy(g_hbm, g_v, sem).wait()
    pltpu.async_copy(b_hbm, b_v, sem).wait()
    liota = lax.iota(jnp.int32, 16)

    pcol = [plsc.load_gather(p_v, [liota, jnp.full((16,), c, jnp.int32)])
            for c in range(DIM)]
    gcol = [plsc.load_gather(g_v, [liota, jnp.full((16,), c, jnp.int32)])
            for c in range(DIM)]
    bcol = [plsc.load_gather(b_v, [liota, jnp.full((16,), c, jnp.int32)])
            for c in range(DIM)]

    def group_body(g, carry):
        gsplat = jnp.full((16,), g, jnp.int32)
        x = []
        for c in range(DIM):
            wsc = plsc.load_gather(w_v, [gsplat, jnp.full((16,), c, jnp.int32)])
            x.append(wsc + pcol[c])
        s = x[0]
        for c in range(1, DIM):
            s = s + x[c]
        mean = s * (1.0 / DIM)
        d0 = x[0] - mean
        s2 = d0 * d0
        for c in range(1, DIM):
            dc = x[c] - mean
            s2 = s2 + dc * dc
        inv = _rsqrt_newton(s2 * (1.0 / DIM) + EPS)
        wbase = g * (16 * DIM)
        for c in range(DIM):
            val = (x[c] - mean) * inv * gcol[c] + bcol[c]
            stage_v[pl.ds(wbase + c * 16, 16)] = val
        return carry

    lax.fori_loop(0, VPW, group_body, 0)
    pltpu.sync_copy(stage_v, tab_hbm.at[pl.ds(v0 * (16 * DIM), VPW * 16 * DIM)])


_sc_table = functools.partial(
    pl.kernel,
    out_type=jax.ShapeDtypeStruct((TWORDS,), jnp.float32),
    mesh=plsc.VectorSubcoreMesh(core_axis_name="c", subcore_axis_name="s"),
    scratch_types=[
        pltpu.VMEM((VPW, DIM), jnp.float32),
        pltpu.VMEM((MAX_POS, DIM), jnp.float32),
        pltpu.VMEM((16, DIM), jnp.float32),
        pltpu.VMEM((16, DIM), jnp.float32),
        pltpu.VMEM((VPW * 16 * DIM,), jnp.float32),
        pltpu.SemaphoreType.DMA,
    ],
    compiler_params=pltpu.CompilerParams(
        use_tc_tiling_on_sc=True, needs_layout_passes=False),
)(_sc_table_body)


def _sc_gather_body(tab_hbm, ids_hbm, out_hbm, tab_v, ids_v, idx_v, buf_v,
                    sem_t, sem_i, sem_o):
    wid = lax.axis_index("s") * NC + lax.axis_index("c")
    base = wid * RPW
    bbase = wid * BPW
    tdesc = pltpu.async_copy(tab_hbm, tab_v, sem_t)
    liota = lax.iota(jnp.int32, 16)

    dimsplat = jnp.full((16,), DIM, jnp.int32)
    pats = []
    for j in range(DIM):
        p = liota + (j * 16)
        rvec = lax.div(p, dimsplat)
        cvec = p - rvec * DIM
        pats.append((rvec, cvec, cvec * 16))

    nchk = BPW // IDC
    idescs = {0: pltpu.async_copy(
        ids_hbm.at[pl.ds(bbase, IDC), :], ids_v.at[0], sem_i)}
    tdesc.wait()
    odescs = {}
    for q in range(nchk):
        par_i = q & 1
        if q + 1 < nchk:
            idescs[q + 1] = pltpu.async_copy(
                ids_hbm.at[pl.ds(bbase + (q + 1) * IDC, IDC), :],
                ids_v.at[1 - par_i], sem_i)
        idescs[q].wait()

        def fidx_body(j, carry):
            idx_v[pl.ds((q * IDC + j) * 16, 16)] = \
                ids_v[par_i, j, :] * (L * DIM) + liota
            return carry

        lax.fori_loop(0, IDC, fidx_body, 0)

        for k in range(NQPC):
            h = q * NQPC + k
            par = h & 1
            if h >= 2:
                odescs[h - 2].wait()

            def pack_body(g, carry):
                r0 = h * QROWS + g * 16
                for j in range(DIM):
                    rvec, cvec, cvec16 = pats[j]
                    wvec = plsc.load_gather(idx_v, [r0 + rvec]) + cvec16
                    v = plsc.load_gather(tab_v, [wvec])
                    plsc.store_scatter(buf_v.at[par], [g * 16 + rvec, cvec], v)
                return carry

            lax.fori_loop(0, QROWS // 16, pack_body, 0)
            odescs[h] = pltpu.async_copy(
                buf_v.at[par],
                out_hbm.at[pl.ds(base + h * QROWS, QROWS), :], sem_o)
    odescs[NQ - 2].wait()
    odescs[NQ - 1].wait()


_sc_gather = functools.partial(
    pl.kernel,
    out_type=jax.ShapeDtypeStruct((ROWS, DIM), jnp.float32),
    mesh=plsc.VectorSubcoreMesh(core_axis_name="c", subcore_axis_name="s"),
    scratch_types=[
        pltpu.VMEM((TWORDS,), jnp.float32),
        pltpu.VMEM((2, IDC, L), jnp.int32),
        pltpu.VMEM((RPW,), jnp.int32),
        pltpu.VMEM((2, QROWS, DIM), jnp.float32),
        pltpu.SemaphoreType.DMA,
        pltpu.SemaphoreType.DMA,
        pltpu.SemaphoreType.DMA,
    ],
    compiler_params=pltpu.CompilerParams(
        use_tc_tiling_on_sc=True, needs_layout_passes=False),
)(_sc_gather_body)


def kernel(input_ids, word_emb, pos_emb, ln_weight, ln_bias):
    w_pad = jnp.zeros((VPAD, DIM), jnp.float32).at[:VOCAB].set(word_emb)
    g16 = jnp.tile(ln_weight.reshape(1, DIM), (16, 1))
    b16 = jnp.tile(ln_bias.reshape(1, DIM), (16, 1))
    table = _sc_table(w_pad, pos_emb, g16, b16)
    out = _sc_gather(table, input_ids)
    return out.reshape(B, L, DIM)
